# baseline (speedup 1.0000x reference)
import functools

import jax
import jax.numpy as jnp
from jax import lax
from jax.experimental import pallas as pl
from jax.experimental.pallas import tpu as pltpu
from jax.experimental.pallas import tpu_sc as plsc

B, C, NNEG, V, D = 16384, 20, 10, 1000000, 64
K = C + NNEG
NW = 32
SPW = B // NW
GS = 16
NG = SPW // GS
RPG = GS * K
IDX_CHUNK = 120
NCHUNK = RPG // IDX_CHUNK


def _gather_start(table_hbm, idx_slice, dst_slice, sem):
    pltpu.async_copy(table_hbm.at[idx_slice], dst_slice, sem)


def _gather_wait(table_hbm, idx_slice, dst_slice, sem):
    pltpu.make_async_copy(table_hbm.at[idx_slice], dst_slice, sem).wait()


def _sc_scores_body(idx_hbm, inv_hbm, table_hbm, out_hbm,
                    idx_v, inv_v, rows0, rows1, scores_v, sem0, sem1):
    wid = lax.axis_index("s") * 2 + lax.axis_index("c")
    base = wid * SPW

    pltpu.sync_copy(idx_hbm.at[pl.ds(base * K, SPW * K)], idx_v)
    pltpu.sync_copy(inv_hbm.at[pl.ds(base * D, SPW * D)], inv_v)

    riota = lax.iota(jnp.int32, 16) * K
    viota = lax.iota(jnp.int32, 16) * D

    bufs = (rows0, rows1)
    sems = (sem0, sem1)

    def issue(g):
        buf, sem = bufs[g % 2], sems[g % 2]
        for c in range(NCHUNK):
            _gather_start(
                table_hbm,
                idx_v.at[pl.ds(g * RPG + c * IDX_CHUNK, IDX_CHUNK)],
                buf.at[pl.ds(c * IDX_CHUNK, IDX_CHUNK), :],
                sem)

    def drain(g):
        buf, sem = bufs[g % 2], sems[g % 2]
        for c in range(NCHUNK):
            _gather_wait(
                table_hbm,
                idx_v.at[pl.ds(g * RPG + c * IDX_CHUNK, IDX_CHUNK)],
                buf.at[pl.ds(c * IDX_CHUNK, IDX_CHUNK), :],
                sem)

    issue(0)
    for g in range(NG):
        if g + 1 < NG:
            issue(g + 1)
        drain(g)
        buf = bufs[g % 2]

        JB = 10
        for j0 in range(0, K, JB):
            def d_body(d, accs, g=g, buf=buf, j0=j0):
                ind = plsc.load_gather(inv_v, [viota + (g * (GS * D) + d)])
                col = jnp.full((16,), d, jnp.int32)
                return tuple(
                    accs[i] + ind * plsc.load_gather(buf, [riota + (j0 + i), col])
                    for i in range(JB))

            accs = lax.fori_loop(
                0, D, d_body,
                tuple(jnp.zeros((16,), jnp.float32) for _ in range(JB)))
            for i in range(JB):
                j = j0 + i
                scores_v[j, pl.ds(g * GS, GS)] = accs[i] if j < C else -accs[i]

    pltpu.sync_copy(scores_v, out_hbm.at[:, pl.ds(base, SPW)])


@jax.jit
def _sc_scores(idx_flat, inv_flat, table):
    mesh = plsc.VectorSubcoreMesh(core_axis_name="c", subcore_axis_name="s")
    return pl.kernel(
        _sc_scores_body,
        mesh=mesh,
        compiler_params=pltpu.CompilerParams(
            needs_layout_passes=False, use_tc_tiling_on_sc=False),
        out_type=jax.ShapeDtypeStruct((K, B), jnp.float32),
        scratch_types=[
            pltpu.VMEM((SPW * K,), jnp.int32),
            pltpu.VMEM((SPW * D,), jnp.float32),
            pltpu.VMEM((RPG, D), jnp.float32),
            pltpu.VMEM((RPG, D), jnp.float32),
            pltpu.VMEM((K, SPW), jnp.float32),
            pltpu.SemaphoreType.DMA,
            pltpu.SemaphoreType.DMA,
        ],
    )(idx_flat, inv_flat, table)


def _tc_loss_body(scores_ref, out_ref):
    x = scores_ref[...]
    ls = jnp.minimum(x, 0.0) - jnp.log1p(jnp.exp(-jnp.abs(x)))
    out_ref[0, 0] = jnp.sum(ls) * (-1.0 / B)


@jax.jit
def _tc_loss(scores2d):
    return pl.pallas_call(
        _tc_loss_body,
        out_shape=jax.ShapeDtypeStruct((1, 1), jnp.float32),
        out_specs=pl.BlockSpec(memory_space=pltpu.SMEM),
    )(scores2d)


def kernel(in_vectors, contexts, neg_contexts, out_emb):
    idx = jnp.concatenate([contexts, neg_contexts], axis=1).reshape(-1)
    inv = in_vectors.reshape(B * D)
    scores = _sc_scores(idx, inv, out_emb)
    loss = _tc_loss(scores.reshape(K * B // 128, 128))
    return loss[0, 0]


# speedup vs baseline: 1.0596x; 1.0596x over previous
import jax
import jax.numpy as jnp
from jax import lax
from jax.experimental import pallas as pl
from jax.experimental.pallas import tpu as pltpu
from jax.experimental.pallas import tpu_sc as plsc

B, C, NNEG, V, D = 16384, 20, 10, 1000000, 64
K = C + NNEG
KP = 32
NW = 32
SPW = B // NW
GS = 16
NG = SPW // GS
JB = 10
NJB = K // JB
RPU = GS * JB
IDX_CHUNK = 80
NCHUNK = RPU // IDX_CHUNK


def _gather_start(table_hbm, idx_slice, dst_slice, sem):
    pltpu.async_copy(table_hbm.at[idx_slice], dst_slice, sem)


def _gather_wait(table_hbm, idx_slice, dst_slice, sem):
    pltpu.make_async_copy(table_hbm.at[idx_slice], dst_slice, sem).wait()


def _sc_scores_body(ctx_hbm, neg_hbm, inv_hbm, table_hbm, out_hbm,
                    ctx_v, neg_v, ilist_v, inv_v, rows0, rows1, scores_v,
                    sem0, sem1):
    wid = lax.axis_index("s") * 2 + lax.axis_index("c")
    base = wid * SPW

    pltpu.sync_copy(ctx_hbm.at[pl.ds(base * C, SPW * C)], ctx_v)
    pltpu.sync_copy(neg_hbm.at[pl.ds(base * NNEG, SPW * NNEG)], neg_v)
    pltpu.sync_copy(inv_hbm.at[pl.ds(base * D, SPW * D)], inv_v)

    iota = lax.iota(jnp.int32, 16)
    iotaC = iota * C
    iotaN = iota * NNEG
    iotaJ = iota * JB
    iotaD = iota * D

    for j in range(K):
        jb, jj = divmod(j, JB)

        def s_body(s, _, j=j, jb=jb, jj=jj):
            if j < C:
                raw = plsc.load_gather(ctx_v, [iotaC + (s * (GS * C) + j)])
            else:
                raw = plsc.load_gather(neg_v, [iotaN + (s * (GS * NNEG) + (j - C))])
            tgt = iotaJ + (s * (NJB * RPU) + jb * RPU + jj)
            plsc.store_scatter(ilist_v, [tgt], raw >> 1)
            return 0

        lax.fori_loop(0, NG, s_body, 0)

    bufs = (rows0, rows1)
    sems = (sem0, sem1)

    def issue(u, par):
        buf, sem = bufs[par], sems[par]
        for c in range(NCHUNK):
            _gather_start(
                table_hbm,
                ilist_v.at[pl.ds(u * RPU + c * IDX_CHUNK, IDX_CHUNK)],
                buf.at[pl.ds(c * IDX_CHUNK, IDX_CHUNK), :],
                sem)

    def drain(u, par):
        buf, sem = bufs[par], sems[par]
        for c in range(NCHUNK):
            _gather_wait(
                table_hbm,
                ilist_v.at[pl.ds(u * RPU + c * IDX_CHUNK, IDX_CHUNK)],
                buf.at[pl.ds(c * IDX_CHUNK, IDX_CHUNK), :],
                sem)

    pad = jnp.full((16,), 1e9, jnp.float32)
    for r in range(K, KP):
        for s in range(NG):
            scores_v[r, pl.ds(s * GS, GS)] = pad

    zero = jnp.zeros((16,), jnp.float32)

    def unit(g, jb, buf):
        pbs, ridx = [], []
        for i in range(JB):
            j = jb * JB + i
            if j < C:
                raw = plsc.load_gather(ctx_v, [iotaC + (g * (GS * C) + j)])
            else:
                raw = plsc.load_gather(neg_v, [iotaN + (g * (GS * NNEG) + (j - C))])
            pbs.append((raw & 1) << 6)
            ridx.append(iotaJ + i)

        @plsc.parallel_loop(0, D, unroll=2, carry=(zero,) * JB)
        def accs(d, accs):
            ind = plsc.load_gather(inv_v, [iotaD + (g * (GS * D) + d)])
            return tuple(
                accs[i] + ind * plsc.load_gather(buf, [ridx[i], pbs[i] + d])
                for i in range(JB))

        for i in range(JB):
            j = jb * JB + i
            scores_v[j, pl.ds(g * GS, GS)] = accs[i] if j < C else -accs[i]

    issue(0, 0)

    def g_pair(p, _):
        gbase = p * 2
        for k in range(2 * NJB):
            u = gbase * NJB + k
            g, jb = gbase + k // NJB, k % NJB
            if k < 2 * NJB - 1:
                issue(u + 1, (k + 1) % 2)
            else:
                @pl.when(p < NG // 2 - 1)
                def _():
                    issue(u + 1, 0)
            drain(u, k % 2)
            unit(g, jb, bufs[k % 2])
        return 0

    lax.fori_loop(0, NG // 2, g_pair, 0)

    pltpu.sync_copy(scores_v, out_hbm.at[:, pl.ds(base, SPW)])


@jax.jit
def _sc_scores(ctx_flat, neg_flat, inv_flat, table2):
    mesh = plsc.VectorSubcoreMesh(core_axis_name="c", subcore_axis_name="s")
    return pl.kernel(
        _sc_scores_body,
        mesh=mesh,
        compiler_params=pltpu.CompilerParams(needs_layout_passes=False),
        out_type=jax.ShapeDtypeStruct((KP, B), jnp.float32),
        scratch_types=[
            pltpu.VMEM((SPW * C,), jnp.int32),
            pltpu.VMEM((SPW * NNEG,), jnp.int32),
            pltpu.VMEM((SPW * K,), jnp.int32),
            pltpu.VMEM((SPW * D,), jnp.float32),
            pltpu.VMEM((RPU, 128), jnp.float32),
            pltpu.VMEM((RPU, 128), jnp.float32),
            pltpu.VMEM((KP, SPW), jnp.float32),
            pltpu.SemaphoreType.DMA,
            pltpu.SemaphoreType.DMA,
        ],
    )(ctx_flat, neg_flat, inv_flat, table2)


def _tc_loss_body(scores_ref, out_ref):
    x = scores_ref[...]
    ls = jnp.minimum(x, 0.0) - jnp.log1p(jnp.exp(-jnp.abs(x)))
    out_ref[0, 0] = jnp.sum(ls) * (-1.0 / B)


@jax.jit
def _tc_loss(scores2d):
    return pl.pallas_call(
        _tc_loss_body,
        out_shape=jax.ShapeDtypeStruct((1, 1), jnp.float32),
        out_specs=pl.BlockSpec(memory_space=pltpu.SMEM),
    )(scores2d)


def kernel(in_vectors, contexts, neg_contexts, out_emb):
    table2 = out_emb.reshape(V // 2, 2 * D)
    scores = _sc_scores(contexts.reshape(-1), neg_contexts.reshape(-1),
                        in_vectors.reshape(-1), table2)
    loss = _tc_loss(scores)
    return loss[0, 0]


# speedup vs baseline: 1.7812x; 1.6810x over previous
import jax
import jax.numpy as jnp
from jax import lax
from jax.experimental import pallas as pl
from jax.experimental.pallas import tpu as pltpu
from jax.experimental.pallas import tpu_sc as plsc

B, C, NNEG, V, D = 16384, 20, 10, 1000000, 64
K = C + NNEG
KP = 32
NW = 32
SPW = B // NW
GS = 16
NG = SPW // GS
JB = 10
NJB = K // JB
RPU = GS * JB
IDX_CHUNK = 80
NCHUNK = RPU // IDX_CHUNK


def _gather_start(table_hbm, idx_slice, dst_slice, sem):
    pltpu.async_copy(table_hbm.at[idx_slice], dst_slice, sem)


def _gather_wait(table_hbm, idx_slice, dst_slice, sem):
    pltpu.make_async_copy(table_hbm.at[idx_slice], dst_slice, sem).wait()


def _sc_scores_body(ctx_hbm, neg_hbm, inv_hbm, table_hbm, out_hbm,
                    ctx_v, neg_v, ilist_v, inv_v, rows0, rows1, scores_v,
                    sem0, sem1):
    wid = lax.axis_index("s") * 2 + lax.axis_index("c")
    base = wid * SPW

    pltpu.sync_copy(ctx_hbm.at[pl.ds(base * C, SPW * C)], ctx_v)
    pltpu.sync_copy(neg_hbm.at[pl.ds(base * NNEG, SPW * NNEG)], neg_v)
    pltpu.sync_copy(inv_hbm.at[pl.ds(base * D, SPW * D)], inv_v)

    iota = lax.iota(jnp.int32, 16)
    iotaC = iota * C
    iotaN = iota * NNEG
    iotaJ = iota * JB
    iotaD = iota * D

    for j in range(K):
        jb, jj = divmod(j, JB)

        def s_body(s, _, j=j, jb=jb, jj=jj):
            if j < C:
                raw = plsc.load_gather(ctx_v, [iotaC + (s * (GS * C) + j)])
            else:
                raw = plsc.load_gather(neg_v, [iotaN + (s * (GS * NNEG) + (j - C))])
            tgt = iotaJ + (s * (NJB * RPU) + jb * RPU + jj)
            plsc.store_scatter(ilist_v, [tgt], raw)
            return 0

        lax.fori_loop(0, NG, s_body, 0)

    bufs = (rows0, rows1)
    sems = (sem0, sem1)

    def issue(u, par):
        buf, sem = bufs[par], sems[par]
        for c in range(NCHUNK):
            _gather_start(
                table_hbm,
                ilist_v.at[pl.ds(u * RPU + c * IDX_CHUNK, IDX_CHUNK)],
                buf.at[pl.ds(c * IDX_CHUNK, IDX_CHUNK), :],
                sem)

    def drain(u, par):
        buf, sem = bufs[par], sems[par]
        for c in range(NCHUNK):
            _gather_wait(
                table_hbm,
                ilist_v.at[pl.ds(u * RPU + c * IDX_CHUNK, IDX_CHUNK)],
                buf.at[pl.ds(c * IDX_CHUNK, IDX_CHUNK), :],
                sem)

    pad = jnp.full((16,), 1e9, jnp.float32)
    for r in range(K, KP):
        for s in range(NG):
            scores_v[r, pl.ds(s * GS, GS)] = pad

    zero = jnp.zeros((16,), jnp.float32)

    def unit(g, jb, buf):
        ridx = [iotaJ + i for i in range(JB)]

        @plsc.parallel_loop(0, D, unroll=2, carry=(zero,) * JB)
        def accs(d, accs):
            col = (iota + d) & (D - 1)
            ind = plsc.load_gather(inv_v, [iotaD + g * (GS * D) + col])
            return tuple(
                accs[i] + ind * plsc.load_gather(buf, [ridx[i], col])
                for i in range(JB))

        for i in range(JB):
            j = jb * JB + i
            scores_v[j, pl.ds(g * GS, GS)] = accs[i] if j < C else -accs[i]

    issue(0, 0)

    def g_pair(p, _):
        gbase = p * 2
        for k in range(2 * NJB):
            u = gbase * NJB + k
            g, jb = gbase + k // NJB, k % NJB
            if k < 2 * NJB - 1:
                issue(u + 1, (k + 1) % 2)
            else:
                @pl.when(p < NG // 2 - 1)
                def _():
                    issue(u + 1, 0)
            drain(u, k % 2)
            unit(g, jb, bufs[k % 2])
        return 0

    lax.fori_loop(0, NG // 2, g_pair, 0)

    pltpu.sync_copy(scores_v, out_hbm.at[:, pl.ds(base, SPW)])


@jax.jit
def _sc_scores(ctx_flat, neg_flat, inv_flat, table2):
    mesh = plsc.VectorSubcoreMesh(core_axis_name="c", subcore_axis_name="s")
    return pl.kernel(
        _sc_scores_body,
        mesh=mesh,
        compiler_params=pltpu.CompilerParams(needs_layout_passes=False),
        out_type=jax.ShapeDtypeStruct((KP, B), jnp.float32),
        scratch_types=[
            pltpu.VMEM((SPW * C,), jnp.int32),
            pltpu.VMEM((SPW * NNEG,), jnp.int32),
            pltpu.VMEM((SPW * K,), jnp.int32),
            pltpu.VMEM((SPW * D,), jnp.float32),
            pltpu.VMEM((RPU, 128), jnp.float32),
            pltpu.VMEM((RPU, 128), jnp.float32),
            pltpu.VMEM((KP, SPW), jnp.float32),
            pltpu.SemaphoreType.DMA,
            pltpu.SemaphoreType.DMA,
        ],
    )(ctx_flat, neg_flat, inv_flat, table2)


def _tc_loss_body(scores_ref, out_ref):
    x = scores_ref[...]
    ls = jnp.minimum(x, 0.0) - jnp.log1p(jnp.exp(-jnp.abs(x)))
    out_ref[0, 0] = jnp.sum(ls) * (-1.0 / B)


@jax.jit
def _tc_loss(scores2d):
    return pl.pallas_call(
        _tc_loss_body,
        out_shape=jax.ShapeDtypeStruct((1, 1), jnp.float32),
        out_specs=pl.BlockSpec(memory_space=pltpu.SMEM),
    )(scores2d)


def kernel(in_vectors, contexts, neg_contexts, out_emb):
    tablep = jnp.pad(out_emb, ((0, 0), (0, D)))
    scores = _sc_scores(contexts.reshape(-1), neg_contexts.reshape(-1),
                        in_vectors.reshape(-1), tablep)
    loss = _tc_loss(scores)
    return loss[0, 0]
